# baseline (speedup 1.0000x reference)
import functools
import math

import jax
import jax.numpy as jnp
from jax import lax
from jax.experimental import pallas as pl
from jax.experimental.pallas import tpu as pltpu
from jax.experimental.pallas import tpu_sc as plsc

NC, NS = 2, 16
NW = NC * NS
DEG = 6.0
_SQRT_HALF = math.sqrt(0.5)


def _gelu(x):
    return 0.5 * x * (1.0 + lax.erf(x * _SQRT_HALF))


def _gn_stats(t, c, groups=8):
    n = t.shape[0]
    xg = t.reshape(n, groups, c // groups)
    mu = xg.mean(-1, keepdims=True)
    var = xg.var(-1, keepdims=True)
    return ((xg - mu) * lax.rsqrt(var + 1e-5)).reshape(n, c)


@functools.lru_cache(maxsize=None)
def _project_fn(n, cin, cout, bn):
    def body(x_ref, w_ref, o_ref):
        x = x_ref[...]
        for t in range(7):
            o_ref[t] = jnp.dot(x, w_ref[t], preferred_element_type=jnp.float32)

    return pl.pallas_call(
        body,
        grid=(n // bn,),
        in_specs=[pl.BlockSpec((bn, cin), lambda i: (i, 0)),
                  pl.BlockSpec((7, cin, cout), lambda i: (0, 0, 0))],
        out_specs=pl.BlockSpec((7, bn, cout), lambda i: (0, i, 0)),
        out_shape=jax.ShapeDtypeStruct((7, n, cout), jnp.float32),
    )


@functools.lru_cache(maxsize=None)
def _post_fn(n, c, bn, residual):
    def body(p_ref, nt_ref, b_ref, ntab_ref, g_ref, beta_ref, *rest):
        if residual:
            x_ref, o_ref = rest
        else:
            (o_ref,) = rest
        p = (p_ref[0] + p_ref[1]) / DEG
        nt = nt_ref[...]
        oh = (nt == lax.broadcasted_iota(jnp.int32, (bn, 9), 1)
              ).astype(jnp.float32)
        t = p + b_ref[...] + jnp.dot(oh, ntab_ref[...],
                                     preferred_element_type=jnp.float32, precision=lax.Precision.HIGHEST)
        xn = _gn_stats(t, c)
        h = xn * g_ref[...] + beta_ref[...]
        if residual:
            o_ref[...] = _gelu(x_ref[...] + h)
        else:
            o_ref[...] = _gelu(h)

    in_specs = [pl.BlockSpec((2, bn, c), lambda i: (0, i, 0)),
                pl.BlockSpec((bn, 1), lambda i: (i, 0)),
                pl.BlockSpec((1, c), lambda i: (0, 0)),
                pl.BlockSpec((9, c), lambda i: (0, 0)),
                pl.BlockSpec((1, c), lambda i: (0, 0)),
                pl.BlockSpec((1, c), lambda i: (0, 0))]
    if residual:
        in_specs.append(pl.BlockSpec((bn, c), lambda i: (i, 0)))
    return pl.pallas_call(
        body,
        grid=(n // bn,),
        in_specs=in_specs,
        out_specs=pl.BlockSpec((bn, c), lambda i: (i, 0)),
        out_shape=jax.ShapeDtypeStruct((n, c), jnp.float32),
    )


@functools.lru_cache(maxsize=None)
def _down_fn(nout, cin, cout, bn):
    def body(x_ref, w_ref, b_ref, o_ref):
        xm = jnp.mean(x_ref[...], axis=1)
        o_ref[...] = jnp.dot(xm, w_ref[...],
                             preferred_element_type=jnp.float32) + b_ref[...]

    return pl.pallas_call(
        body,
        grid=(nout // bn,),
        in_specs=[pl.BlockSpec((bn, 8, cin), lambda i: (i, 0, 0)),
                  pl.BlockSpec((cin, cout), lambda i: (0, 0)),
                  pl.BlockSpec((1, cout), lambda i: (0, 0))],
        out_specs=pl.BlockSpec((bn, cout), lambda i: (i, 0)),
        out_shape=jax.ShapeDtypeStruct((nout, cout), jnp.float32),
    )


@functools.lru_cache(maxsize=None)
def _upsample_fn(nin, cin, cout, bn, with_skip):
    def body(x_ref, w_ref, b_ref, *rest):
        y = jnp.dot(x_ref[...], w_ref[...],
                    preferred_element_type=jnp.float32) + b_ref[...]
        y3 = y[:, None, :]
        if with_skip:
            skip_ref, o_ref = rest
            o_ref[...] = y3 + skip_ref[...]
        else:
            (o_ref,) = rest
            o_ref[...] = jnp.broadcast_to(y3, (bn, 8, cout))

    in_specs = [pl.BlockSpec((bn, cin), lambda i: (i, 0)),
                pl.BlockSpec((cin, cout), lambda i: (0, 0)),
                pl.BlockSpec((1, cout), lambda i: (0, 0))]
    if with_skip:
        in_specs.append(pl.BlockSpec((bn, 8, cout), lambda i: (i, 0, 0)))
    return pl.pallas_call(
        body,
        grid=(nin // bn,),
        in_specs=in_specs,
        out_specs=pl.BlockSpec((bn, 8, cout), lambda i: (i, 0, 0)),
        out_shape=jax.ShapeDtypeStruct((nin, 8, cout), jnp.float32),
    )


@functools.lru_cache(maxsize=None)
def _pred_fn(n, cin, mid, cout, bn):
    def body(x_ref, w1_ref, b1_ref, g_ref, beta_ref, w2_ref, b2_ref, o_ref):
        h = jnp.dot(x_ref[...], w1_ref[...],
                    preferred_element_type=jnp.float32) + b1_ref[...]
        hn = _gn_stats(h, mid)
        h = _gelu(hn * g_ref[...] + beta_ref[...])
        o_ref[...] = jnp.dot(h, w2_ref[...],
                             preferred_element_type=jnp.float32) + b2_ref[...]

    return pl.pallas_call(
        body,
        grid=(n // bn,),
        in_specs=[pl.BlockSpec((bn, cin), lambda i: (i, 0)),
                  pl.BlockSpec((cin, mid), lambda i: (0, 0)),
                  pl.BlockSpec((1, mid), lambda i: (0, 0)),
                  pl.BlockSpec((1, mid), lambda i: (0, 0)),
                  pl.BlockSpec((1, mid), lambda i: (0, 0)),
                  pl.BlockSpec((mid, cout), lambda i: (0, 0)),
                  pl.BlockSpec((1, cout), lambda i: (0, 0))],
        out_specs=pl.BlockSpec((bn, cout), lambda i: (i, 0)),
        out_shape=jax.ShapeDtypeStruct((n, cout), jnp.float32),
    )


@functools.lru_cache(maxsize=None)
def _agg_fn(n, cout, e_pad, b, chunks):
    nchunks = len(chunks)
    max_ch = max(ch for _, ch in chunks)
    R = max_ch + 16
    ept = e_pad // NW
    nblk = ept // b

    mesh = plsc.VectorSubcoreMesh(core_axis_name="c", subcore_axis_name="s")

    @functools.partial(
        pl.kernel,
        out_type=jax.ShapeDtypeStruct((2, n, cout), jnp.float32),
        mesh=mesh,
        scratch_types=[
            pltpu.VMEM((b,), jnp.int32),
            pltpu.VMEM((b,), jnp.int32),
            pltpu.VMEM((b,), jnp.int32),
            pltpu.VMEM((b,), jnp.int32),
            pltpu.VMEM((b,), jnp.int32),
            pltpu.VMEM((b, cout), jnp.float32),
            pltpu.VMEM((256, cout), jnp.float32),
            pltpu.VMEM_SHARED((R, cout), jnp.float32),
            pltpu.SemaphoreType.DMA,
        ],
        compiler_params=pltpu.CompilerParams(use_tc_tiling_on_sc=False),
    )
    def kfn(y_hbm, src_hbm, dst_hbm, et_hbm, out_hbm,
            src_v, dst_v, et_v, gidx_v, didx_v, rows_v, zstage_v,
            agg_sh, sem):
        c = lax.axis_index("c")
        s = lax.axis_index("s")
        wid = c * NS + s
        tile_base = wid * ept
        zstage_v[...] = jnp.zeros_like(zstage_v)

        for c0, ch in chunks:
            zr = (ch + 16) // 16
            zb = min(zr, 256)
            zfull, zrem = zr // zb, zr % zb
            orps = ch // 16
            for kk in range(zfull):
                pltpu.sync_copy(zstage_v.at[pl.ds(0, zb)],
                                agg_sh.at[pl.ds(s * zr + kk * zb, zb)])
            if zrem:
                pltpu.sync_copy(zstage_v.at[pl.ds(0, zrem)],
                                agg_sh.at[pl.ds(s * zr + zfull * zb, zrem)])
            plsc.subcore_barrier()

            def blk_body(blk, carry):
                base = tile_base + blk * b
                pltpu.sync_copy(src_hbm.at[pl.ds(base, b)], src_v)
                pltpu.sync_copy(dst_hbm.at[pl.ds(base, b)], dst_v)
                pltpu.sync_copy(et_hbm.at[pl.ds(base, b)], et_v)
                for k in range(b // 16):
                    sl = pl.ds(k * 16, 16)
                    sv = src_v[sl]
                    ev = et_v[sl]
                    dv = dst_v[sl]
                    g = ev * n + sv
                    if nchunks > 1:
                        m = (dv >= c0) & (dv < c0 + ch)
                        d = jnp.where(m, dv - c0, ch)
                    else:
                        d = dv
                    gidx_v[sl] = g
                    didx_v[sl] = d
                pltpu.async_copy(y_hbm.at[gidx_v], rows_v, sem).wait()
                pltpu.sync_copy(rows_v, agg_sh.at[didx_v], add=True)
                return carry

            lax.fori_loop(0, nblk, blk_body, 0)
            plsc.subcore_barrier()
            pltpu.sync_copy(
                agg_sh.at[pl.ds(s * orps, orps)],
                out_hbm.at[c, pl.ds(c0 + s * orps, orps)])
            plsc.subcore_barrier()

    return kfn


def _bn_for(n):
    return min(n, 2048)


def _graph_conv_parts(x, W, g):
    src, dst, et, n, c = g["src"], g["dst"], g["et"], g["n"], g["c"]
    y = _project_fn(n, c, c, _bn_for(n))(x, W)
    y2 = y.reshape(7 * n, c)
    agg = _agg_fn(n, c, g["e_pad"], g["b"], g["chunks"])
    return agg(y2, src, dst, et)


def _resblk(x, g, p):
    n, c = g["n"], g["c"]
    bn = _bn_for(n)
    parts1 = _graph_conv_parts(x, p["c1"]["W"], g)
    h = _post_fn(n, c, bn, False)(
        parts1, g["nt1"], p["c1"]["b"].reshape(1, c), p["c1"]["nt"],
        p["n1"]["g"].reshape(1, c), p["n1"]["b"].reshape(1, c))
    parts2 = _graph_conv_parts(h, p["c2"]["W"], g)
    return _post_fn(n, c, bn, True)(
        parts2, g["nt1"], p["c2"]["b"].reshape(1, c), p["c2"]["nt"],
        p["n2"]["g"].reshape(1, c), p["n2"]["b"].reshape(1, c), x)


def _run_blocks(x, g, ps):
    for p in ps:
        x = _resblk(x, g, p)
    return x


def _downsample(x, p):
    n, c = x.shape
    cout = p["W"].shape[1]
    xr = x.reshape(n // 8, 8, c)
    return _down_fn(n // 8, c, cout, _bn_for(n // 8))(
        xr, p["W"], p["b"].reshape(1, cout))


def _upsample(x, p, skip=None):
    n, c = x.shape
    cout = p["W"].shape[1]
    bn = _bn_for(n)
    if skip is not None:
        sk3 = skip.reshape(n, 8, cout)
        out3 = _upsample_fn(n, c, cout, bn, True)(
            x, p["W"], p["b"].reshape(1, cout), sk3)
    else:
        out3 = _upsample_fn(n, c, cout, bn, False)(
            x, p["W"], p["b"].reshape(1, cout))
    return out3.reshape(8 * n, cout)


def _prediction(x, p):
    n, cin = x.shape
    mid = p["l1"]["W"].shape[1]
    cout = p["l2"]["W"].shape[1]
    return _pred_fn(n, cin, mid, cout, _bn_for(n))(
        x, p["l1"]["W"], p["l1"]["b"].reshape(1, mid),
        p["n"]["g"].reshape(1, mid), p["n"]["b"].reshape(1, mid),
        p["l2"]["W"], p["l2"]["b"].reshape(1, cout))


def _make_graph(edge_index, edge_type, node_type, n, c, chunks):
    e = edge_index.shape[1]
    ept = -(-e // NW)
    ept = max(16, ((ept + 15) // 16) * 16)
    e_pad = ept * NW
    src, dst, et = edge_index[0], edge_index[1], edge_type
    if e_pad != e:
        pad = e_pad - e
        src = jnp.concatenate([src, jnp.zeros((pad,), jnp.int32)])
        dst = jnp.concatenate([dst, jnp.full((pad,), n, jnp.int32)])
        et = jnp.concatenate([et, jnp.zeros((pad,), jnp.int32)])
    b = ept if ept <= 1024 else 1024
    while ept % b:
        b //= 2
    return {"src": src, "dst": dst, "et": et, "n": n, "c": c,
            "e_pad": e_pad, "b": b, "chunks": chunks,
            "nt1": node_type.reshape(n, 1)}


def kernel(data, params, edge_index_lm2, edge_type_lm2, node_type_lm2,
           edge_index_lm1, edge_type_lm1, node_type_lm1,
           edge_index_l0, edge_type_l0, node_type_l0,
           edge_index_lp1, edge_type_lp1, node_type_lp1,
           edge_index_lp2, edge_type_lp2, node_type_lp2):
    g_lm2 = _make_graph(edge_index_lm2, edge_type_lm2, node_type_lm2,
                        64, 256, ((0, 64),))
    g_lm1 = _make_graph(edge_index_lm1, edge_type_lm1, node_type_lm1,
                        512, 128, ((0, 512),))
    g_l0 = _make_graph(edge_index_l0, edge_type_l0, node_type_l0,
                       4096, 64, ((0, 4096),))
    g_lp1 = _make_graph(edge_index_lp1, edge_type_lp1, node_type_lp1,
                        32768, 32, ((0, 32768),))
    g_lp2 = _make_graph(edge_index_lp2, edge_type_lp2, node_type_lp2,
                        262144, 24,
                        ((0, 53248), (53248, 53248), (106496, 53248),
                         (159744, 53248), (212992, 49152)))

    p = params
    e0 = _run_blocks(data, g_l0, p["enc0"])
    x = _downsample(e0, p["down0"])
    e1 = _run_blocks(x, g_lm1, p["enc1"])
    x = _downsample(e1, p["down1"])
    e2 = _run_blocks(x, g_lm2, p["enc2"])
    x = _run_blocks(e2, g_lm2, p["ndec0"])
    x = _upsample(x, p["nup0"], skip=e1)
    x = _run_blocks(x, g_lm1, p["ndec1"])
    x = _upsample(x, p["nup1"], skip=e0)
    x = _run_blocks(x, g_l0, p["ndec2"])
    x = _run_blocks(x, g_l0, p["odec0"])
    s0 = _prediction(x, p["pred0"])
    x = _upsample(x, p["oup0"])
    x = _run_blocks(x, g_lp1, p["odec1"])
    s1 = _prediction(x, p["pred1"])
    x = _upsample(x, p["oup1"])
    x = _run_blocks(x, g_lp2, p["odec2"])
    s2 = _prediction(x, p["pred2"])
    return (s0, s1, s2)
